# Initial kernel scaffold; baseline (speedup 1.0000x reference)
import jax
import jax.numpy as jnp
from jax.experimental import pallas as pl


def kernel(users, pos_items, neg_items, user_biases, item_biases, user_embeddings, item_embeddings):
    raise NotImplementedError("write your pallas kernel here")


# speedup vs baseline: 2.0234x; 2.0234x over previous
import functools

import jax
import jax.numpy as jnp
from jax import lax
from jax.experimental import pallas as pl
from jax.experimental.pallas import tpu as pltpu
from jax.experimental.pallas import tpu_sc as plsc

_B = 16384
_D = 128
_NC = 2
_NS = 16
_NW = _NC * _NS
_BPW = _B // _NW
_CH = 256
_NCHUNK = _BPW // _CH

_mesh = plsc.VectorSubcoreMesh(core_axis_name="c", subcore_axis_name="s")


@functools.partial(
    pl.kernel,
    mesh=_mesh,
    out_type=jax.ShapeDtypeStruct((_B,), jnp.float32),
    scratch_types=[
        pltpu.VMEM((_CH,), jnp.int32),
        pltpu.VMEM((_CH,), jnp.int32),
        pltpu.VMEM((_CH,), jnp.int32),
        pltpu.VMEM((_CH, _D), jnp.float32),
        pltpu.VMEM((_CH, _D), jnp.float32),
        pltpu.VMEM((_CH, _D), jnp.float32),
        pltpu.VMEM((_CH,), jnp.float32),
        pltpu.VMEM((_CH,), jnp.float32),
        pltpu.VMEM((_CH,), jnp.float32),
        pltpu.SemaphoreType.DMA,
    ],
)
def _cplr_sc(users, pos_items, neg_items, item_biases, ue, ie, out,
             idx_u, idx_p, idx_n, rows_u, rows_p, rows_n, pb, nb, out_v,
             sem):
    wid = lax.axis_index("s") * _NC + lax.axis_index("c")
    base = wid * _BPW

    def chunk_body(c, carry):
        off = base + c * _CH
        pltpu.sync_copy(users.at[pl.ds(off, _CH)], idx_u)
        pltpu.sync_copy(pos_items.at[pl.ds(off, _CH)], idx_p)
        pltpu.sync_copy(neg_items.at[pl.ds(off, _CH)], idx_n)
        cu = pltpu.async_copy(ue.at[idx_u], rows_u, sem)
        cp = pltpu.async_copy(ie.at[idx_p], rows_p, sem)
        cn = pltpu.async_copy(ie.at[idx_n], rows_n, sem)
        cb1 = pltpu.async_copy(item_biases.at[idx_p], pb, sem)
        cb2 = pltpu.async_copy(item_biases.at[idx_n], nb, sem)
        cu.wait()
        cp.wait()
        cn.wait()
        cb1.wait()
        cb2.wait()

        lanes = lax.iota(jnp.int32, 16)

        def group_body(g, carry2):
            e0 = g * 16
            tot = jnp.zeros((16,), jnp.float32)
            for i in range(16):
                e = e0 + i
                acc = jnp.zeros((16,), jnp.float32)
                for j in range(_D // 16):
                    u = rows_u[e, pl.ds(j * 16, 16)]
                    p = rows_p[e, pl.ds(j * 16, 16)]
                    n = rows_n[e, pl.ds(j * 16, 16)]
                    acc = acc + u * (p - n)
                parts = [acc[k] for k in range(16)]
                while len(parts) > 1:
                    parts = [parts[k] + parts[k + 1]
                             for k in range(0, len(parts), 2)]
                tot = jnp.where(lanes == i, parts[0], tot)
            pbv = pb[pl.ds(e0, 16)]
            nbv = nb[pl.ds(e0, 16)]
            out_v[pl.ds(e0, 16)] = tot + pbv - nbv
            return carry2

        lax.fori_loop(0, _CH // 16, group_body, 0)
        pltpu.sync_copy(out_v, out.at[pl.ds(off, _CH)])
        return carry

    lax.fori_loop(0, _NCHUNK, chunk_body, 0)


def kernel(users, pos_items, neg_items, user_biases, item_biases,
           user_embeddings, item_embeddings):
    del user_biases
    return _cplr_sc(
        users.astype(jnp.int32),
        pos_items.astype(jnp.int32),
        neg_items.astype(jnp.int32),
        item_biases.reshape(-1),
        user_embeddings,
        item_embeddings,
    )
